# Initial kernel scaffold; baseline (speedup 1.0000x reference)
import jax
import jax.numpy as jnp
from jax.experimental import pallas as pl


def kernel(num_nodes, table):
    raise NotImplementedError("write your pallas kernel here")


# speedup vs baseline: 1.4462x; 1.4462x over previous
import functools

import jax
import jax.numpy as jnp
from jax import lax
from jax.experimental import pallas as pl
from jax.experimental.pallas import tpu as pltpu
from jax.experimental.pallas import tpu_sc as plsc

_N = 512
_D = 128
_K = 10
_T = 2 * _K + 1
_NW = 32
_RPW = _N // _NW
_WIN = _N + _RPW - 1
_CH = 128
_NCH = -(-_WIN // _CH)
_WPAD = _NCH * _CH


def _index_grid():
    w = jnp.arange(_NW, dtype=jnp.int32)[:, None]
    t = jnp.arange(_WPAD, dtype=jnp.int32)[None, :]
    return jnp.clip(t - 5 - _RPW * w, 0, _T - 1).reshape(_NW, _NCH, _CH)


@functools.partial(
    pl.kernel,
    out_type=jax.ShapeDtypeStruct((_N, _N, _D), jnp.float32),
    mesh=plsc.VectorSubcoreMesh(core_axis_name="c", subcore_axis_name="s"),
    scratch_types=[
        pltpu.VMEM((_NCH, _CH), jnp.int32),
        pltpu.VMEM((_WPAD, _D), jnp.float32),
        pltpu.SemaphoreType.DMA,
        pltpu.SemaphoreType.DMA,
    ],
)
def _sc_band_fill(table_hbm, idx_hbm, out_hbm, idx_v, win_v, gsem, wsem):
    wid = lax.axis_index("s") * 2 + lax.axis_index("c")
    pltpu.sync_copy(idx_hbm.at[wid], idx_v)
    gathers = [
        pltpu.async_copy(
            table_hbm.at[idx_v.at[c]], win_v.at[pl.ds(c * _CH, _CH)], gsem
        )
        for c in range(_NCH)
    ]
    for g in gathers:
        g.wait()
    r0 = wid * _RPW
    writes = [
        pltpu.async_copy(
            win_v.at[pl.ds(_RPW - 1 - p, _N)], out_hbm.at[r0 + p], wsem
        )
        for p in range(_RPW)
    ]
    for wr in writes:
        wr.wait()


def kernel(num_nodes, table):
    del num_nodes
    return _sc_band_fill(table, _index_grid())


# speedup vs baseline: 7.4850x; 5.1756x over previous
import functools

import jax
import jax.numpy as jnp
from jax import lax
from jax.experimental import pallas as pl
from jax.experimental.pallas import tpu as pltpu
from jax.experimental.pallas import tpu_sc as plsc

_N = 512
_D = 128
_K = 10
_T = 2 * _K + 1
_B = 2 * _N - 1
_NC = 2
_RPC = _N // _NC
_LO = _N - 11
_FILL = _LO


def _doubling_fill(b_sh, src_row, dst0):
    pltpu.sync_copy(b_sh.at[pl.ds(src_row, 1)], b_sh.at[pl.ds(dst0, 1)])
    cur = 1
    while cur < _FILL:
        n = min(cur, _FILL - cur)
        pltpu.sync_copy(
            b_sh.at[pl.ds(dst0, n)], b_sh.at[pl.ds(dst0 + cur, n)]
        )
        cur += n


@functools.partial(
    pl.kernel,
    out_type=jax.ShapeDtypeStruct((_N, _N, _D), jnp.float32),
    mesh=plsc.ScalarSubcoreMesh(axis_name="c", num_cores=_NC),
    scratch_types=[
        pltpu.VMEM_SHARED((_B, _D), jnp.float32),
        pltpu.SemaphoreType.DMA,
    ],
)
def _sc_band_fill(table_hbm, out_hbm, b_sh, sem):
    cid = lax.axis_index("c")
    pltpu.sync_copy(table_hbm, b_sh.at[pl.ds(_LO, _T)])
    _doubling_fill(b_sh, _LO, 0)
    _doubling_fill(b_sh, _LO + _T - 1, _LO + _T)
    r0 = cid * _RPC

    def issue(i, carry):
        row = r0 + i
        pltpu.async_copy(
            b_sh.at[pl.ds(_N - 1 - row, _N)], out_hbm.at[row], sem
        )
        return carry

    lax.fori_loop(0, _RPC, issue, 0)

    def drain(i, carry):
        pltpu.make_async_copy(
            out_hbm.at[0], b_sh.at[pl.ds(0, _N)], sem
        ).wait()
        return carry

    lax.fori_loop(0, _RPC, drain, 0)


def kernel(num_nodes, table):
    del num_nodes
    return _sc_band_fill(table)


# speedup vs baseline: 8.0407x; 1.0742x over previous
import functools

import jax
import jax.numpy as jnp
from jax import lax
from jax.experimental import pallas as pl
from jax.experimental.pallas import tpu as pltpu
from jax.experimental.pallas import tpu_sc as plsc

_N = 512
_D = 128
_K = 10
_T = 2 * _K + 1
_B = 2 * _N - 1
_NC = 2
_RPC = _N // _NC
_LO = _N - 11
_FILL = _LO


def _fill_round(b_sh, sem, src_row, dst0, have):
    copies = []
    if have == 0:
        for k in range(8):
            copies.append(
                pltpu.async_copy(
                    b_sh.at[pl.ds(src_row, 1)], b_sh.at[pl.ds(dst0 + k, 1)], sem
                )
            )
        return copies, 8
    pos = have
    for _ in range(7):
        if pos >= _FILL:
            break
        n = min(have, _FILL - pos)
        copies.append(
            pltpu.async_copy(
                b_sh.at[pl.ds(dst0, n)], b_sh.at[pl.ds(dst0 + pos, n)], sem
            )
        )
        pos += n
    return copies, pos


@functools.partial(
    pl.kernel,
    out_type=jax.ShapeDtypeStruct((_N, _N, _D), jnp.float32),
    mesh=plsc.ScalarSubcoreMesh(axis_name="c", num_cores=_NC),
    scratch_types=[
        pltpu.VMEM_SHARED((_B, _D), jnp.float32),
        pltpu.SemaphoreType.DMA,
    ],
)
def _sc_band_fill(table_hbm, out_hbm, b_sh, sem):
    cid = lax.axis_index("c")
    pltpu.sync_copy(table_hbm, b_sh.at[pl.ds(_LO, _T)])
    have_l = have_r = 0
    while have_l < _FILL or have_r < _FILL:
        cl, have_l = _fill_round(b_sh, sem, _LO, 0, have_l)
        cr, have_r = _fill_round(b_sh, sem, _LO + _T - 1, _LO + _T, have_r)
        for c in cl + cr:
            c.wait()
    r0 = cid * _RPC

    def issue(i, carry):
        row = r0 + i
        pltpu.async_copy(
            b_sh.at[pl.ds(_N - 1 - row, _N)], out_hbm.at[row], sem
        )
        return carry

    lax.fori_loop(0, _RPC, issue, 0)

    def drain(i, carry):
        pltpu.make_async_copy(
            out_hbm.at[0], b_sh.at[pl.ds(0, _N)], sem
        ).wait()
        return carry

    lax.fori_loop(0, _RPC, drain, 0)


def kernel(num_nodes, table):
    del num_nodes
    return _sc_band_fill(table)
